# Initial kernel scaffold; baseline (speedup 1.0000x reference)
import jax
import jax.numpy as jnp
from jax.experimental import pallas as pl


def kernel(x, edge_index, edge_attr, batch, We, be, eps, W1, b1, W2, b2, ln_g, ln_b, Wg, bg, Wr1, br1, g1, beta1, Wr2, br2, g2, beta2, Wo, bo):
    raise NotImplementedError("write your pallas kernel here")


# speedup vs baseline: 2.1104x; 2.1104x over previous
import functools

import jax
import jax.numpy as jnp
from jax import lax
from jax.experimental import pallas as pl
from jax.experimental.pallas import tpu as pltpu
from jax.experimental.pallas import tpu_sc as plsc

N = 10000
E = 320000
D = 128
ED = 16
G = 256

CHUNK = 128
NCHUNK = E // CHUNK
NWORK = 32
KMAX = (NCHUNK + NWORK - 1) // NWORK
ROWS_PER_TILE = 624


def _dot3(a, b, dims=None):
    ah = a.astype(jnp.bfloat16)
    al = (a - ah.astype(jnp.float32)).astype(jnp.bfloat16)
    bh = b.astype(jnp.bfloat16)
    bl = (b - bh.astype(jnp.float32)).astype(jnp.bfloat16)
    if dims is None:
        def f(u, v):
            return jnp.dot(u, v, preferred_element_type=jnp.float32)
    else:
        def f(u, v):
            return lax.dot_general(u, v, dims,
                                   preferred_element_type=jnp.float32)
    return f(ah, bh) + f(ah, bl) + f(al, bh)


def _dotx(a, b, dims=None):
    a1 = a.astype(jnp.bfloat16)
    ra = a - a1.astype(jnp.float32)
    a2 = ra.astype(jnp.bfloat16)
    a3 = (ra - a2.astype(jnp.float32)).astype(jnp.bfloat16)
    b1 = b.astype(jnp.bfloat16)
    rb = b - b1.astype(jnp.float32)
    b2 = rb.astype(jnp.bfloat16)
    b3 = (rb - b2.astype(jnp.float32)).astype(jnp.bfloat16)
    if dims is None:
        def f(u, v):
            return jnp.dot(u, v, preferred_element_type=jnp.float32)
    else:
        def f(u, v):
            return lax.dot_general(u, v, dims,
                                   preferred_element_type=jnp.float32)
    return ((f(a3, b1) + f(a2, b2) + f(a1, b3))
            + (f(a2, b1) + f(a1, b2)) + f(a1, b1))


def _dotd(a, b):
    return jnp.dot(a, b, preferred_element_type=jnp.float32)

def _edge_lin_body(ea_ref, w_ref, b_ref, o_ref):
    o_ref[...] = (
        _dotd(ea_ref[...], w_ref[...])
        + b_ref[...]
    )


def _edge_lin(edge_attr, w, b):
    EB = 4000
    return pl.pallas_call(
        _edge_lin_body,
        grid=(E // EB,),
        in_specs=[
            pl.BlockSpec((EB, ED), lambda i: (i, 0)),
            pl.BlockSpec((ED, D), lambda i: (0, 0)),
            pl.BlockSpec((1, D), lambda i: (0, 0)),
        ],
        out_specs=pl.BlockSpec((EB, D), lambda i: (i, 0)),
        out_shape=jax.ShapeDtypeStruct((E, D), jnp.float32),
    )(edge_attr, w, b)


def _gine_sc_body(h_hbm, e_hbm, ssrc_hbm, sdst_hbm, perm_hbm, bnd_hbm,
                  out_hbm, sbuf, dbuf, pbuf, bndv, hrows, erows, aggbuf, sem):
    c = lax.axis_index("c")
    s = lax.axis_index("s")
    wid = s * 2 + c
    nw = jnp.where(wid < 30, 312 * wid, 9360 + 320 * (wid - 30))

    pltpu.sync_copy(bnd_hbm, bndv)

    zero16 = jnp.zeros((16,), jnp.float32)

    def zbody(i, carry):
        for j in range(8):
            aggbuf[pl.ds(i * 128 + j * 16, 16)] = zero16
        return carry

    lax.fori_loop(0, 320, zbody, 0)

    eb = bndv[0, pl.ds(wid, 16)][0]
    ee = bndv[0, pl.ds(wid + 1, 16)][0]
    c0 = eb // 128
    c1 = (ee + 127) // 128

    def chunk_body(t, carry):
        pltpu.sync_copy(ssrc_hbm.at[pl.ds(t, 1)], sbuf)
        pltpu.sync_copy(sdst_hbm.at[pl.ds(t, 1)],
                        dbuf.at[:, pl.ds(0, 128)])
        pltpu.sync_copy(perm_hbm.at[pl.ds(t, 1)], pbuf)
        pltpu.async_copy(h_hbm.at[sbuf.at[0]], hrows, sem).wait()
        pltpu.async_copy(e_hbm.at[pbuf.at[0]], erows, sem).wait()
        i_lo = jnp.maximum(eb, t * 128) - t * 128
        i_hi = jnp.minimum(ee, (t + 1) * 128) - t * 128

        def edge_body(li, ec):
            cur = ec[0]
            accs = ec[1]
            nd = dbuf[0, pl.ds(li, 16)][0]
            is_new = nd != cur

            @pl.when(is_new & (cur >= 0))
            def _():
                row = (cur - nw) * 128
                for j in range(8):
                    aggbuf[pl.ds(row + j * 16, 16)] = accs[j]

            new_accs = []
            for j in range(8):
                sl = pl.ds(j * 16, 16)
                msg = jnp.maximum(hrows[li, sl] + erows[li, sl], 0.0)
                new_accs.append(jnp.where(is_new, msg, accs[j] + msg))
            return (nd, tuple(new_accs))

        return lax.fori_loop(i_lo, i_hi, edge_body, carry)

    acc0 = tuple(jnp.zeros((16,), jnp.float32) for _ in range(8))
    fin = lax.fori_loop(c0, c1, chunk_body, (jnp.int32(-1), acc0))
    cur = fin[0]

    @pl.when(cur >= 0)
    def _():
        row = (cur - nw) * 128
        for j in range(8):
            aggbuf[pl.ds(row + j * 16, 16)] = fin[1][j]

    pltpu.sync_copy(aggbuf.at[pl.ds(0, 312 * 128)],
                    out_hbm.at[pl.ds(nw * 128, 312 * 128)])

    @pl.when(wid >= 30)
    def _():
        pltpu.sync_copy(aggbuf.at[pl.ds(312 * 128, 8 * 128)],
                        out_hbm.at[pl.ds((nw + 312) * 128, 8 * 128)])


@functools.cache
def _gine_sc_kernel():
    mesh = plsc.VectorSubcoreMesh(
        core_axis_name="c", subcore_axis_name="s", num_cores=2,
        num_subcores=16)
    return pl.kernel(
        _gine_sc_body,
        out_type=jax.ShapeDtypeStruct((N * D,), jnp.float32),
        mesh=mesh,
        scratch_types=[
            pltpu.VMEM((1, 128), jnp.int32),
            pltpu.VMEM((1, 144), jnp.int32),
            pltpu.VMEM((1, 128), jnp.int32),
            pltpu.VMEM((1, 128), jnp.int32),
            pltpu.VMEM((CHUNK, D), jnp.float32),
            pltpu.VMEM((CHUNK, D), jnp.float32),
            pltpu.VMEM((320 * 128,), jnp.float32),
            pltpu.SemaphoreType.DMA,
        ],
    )


def _gine_sc(h, e, ssrc2, sdst2, perm2, bnd):
    return _gine_sc_kernel()(h, e, ssrc2, sdst2, perm2, bnd).reshape(N, D)


def _node_mlp_body(h_ref, a_ref, eps_ref, w1_ref, b1_ref, w2_ref, b2_ref,
                   g_ref, bb_ref, o_ref):
    h = h_ref[...]
    z = eps_ref[0, 0] * h + a_ref[...]
    t = jnp.maximum(
        _dotd(z, w1_ref[...])
        + b1_ref[...], 0.0)
    z2 = (_dotd(t, w2_ref[...])
          + b2_ref[...])
    mu = jnp.mean(z2, axis=1, keepdims=True)
    var = jnp.mean((z2 - mu) ** 2, axis=1, keepdims=True)
    z2 = (z2 - mu) * lax.rsqrt(var + 1e-5) * g_ref[...] + bb_ref[...]
    o_ref[...] = jnp.maximum(z2 + h, 0.0)


def _node_mlp(h, agg, epsl, w1, b1, w2, b2, g, bb):
    NB = 1000
    return pl.pallas_call(
        _node_mlp_body,
        grid=(N // NB,),
        in_specs=[
            pl.BlockSpec((NB, D), lambda i: (i, 0)),
            pl.BlockSpec((NB, D), lambda i: (i, 0)),
            pl.BlockSpec((1, 1), lambda i: (0, 0)),
            pl.BlockSpec((D, D), lambda i: (0, 0)),
            pl.BlockSpec((1, D), lambda i: (0, 0)),
            pl.BlockSpec((D, D), lambda i: (0, 0)),
            pl.BlockSpec((1, D), lambda i: (0, 0)),
            pl.BlockSpec((1, D), lambda i: (0, 0)),
            pl.BlockSpec((1, D), lambda i: (0, 0)),
        ],
        out_specs=pl.BlockSpec((NB, D), lambda i: (i, 0)),
        out_shape=jax.ShapeDtypeStruct((N, D), jnp.float32),
    )(h, agg, epsl, w1, b1, w2, b2, g, bb)


def _pool_body(h_ref, b_ref, wg_ref, bg_ref, o_ref):
    h = h_ref[...]
    bat = b_ref[...]
    gate = (_dotd(h, wg_ref[...])
            + bg_ref[0, 0])
    gid = lax.broadcasted_iota(jnp.int32, (1, G), 1)
    m = bat == gid
    gm = jnp.max(jnp.where(m, gate, -1e30), axis=0, keepdims=True)
    gmn = jnp.sum(jnp.where(m, gm, 0.0), axis=1, keepdims=True)
    ex = jnp.exp(gate - gmn)
    mex = jnp.where(m, ex, 0.0)
    ones = jnp.ones((h.shape[0], 1), jnp.float32)
    dims = (((0,), (0,)), ((), ()))
    denom = _dotx(mex, ones, dims)
    pooled_u = _dotx(mex, h, dims)
    o_ref[...] = pooled_u / (denom + 1e-12)


def _pool(h, bat2, wg, bg):
    return pl.pallas_call(
        _pool_body,
        out_shape=jax.ShapeDtypeStruct((G, D), jnp.float32),
    )(h, bat2, wg, bg)


def _reg_body(p_ref, wr1_ref, br1_ref, g1_ref, be1_ref, wr2_ref, br2_ref,
              g2_ref, be2_ref, wo_ref, bo_ref, o_ref):
    r = (_dotd(p_ref[...], wr1_ref[...])
         + br1_ref[...])
    mu = jnp.mean(r, axis=0, keepdims=True)
    var = jnp.mean((r - mu) ** 2, axis=0, keepdims=True)
    r = jnp.maximum((r - mu) * lax.rsqrt(var + 1e-5) * g1_ref[...]
                    + be1_ref[...], 0.0)
    r = (_dotd(r, wr2_ref[...])
         + br2_ref[...])
    mu = jnp.mean(r, axis=0, keepdims=True)
    var = jnp.mean((r - mu) ** 2, axis=0, keepdims=True)
    r = jnp.maximum((r - mu) * lax.rsqrt(var + 1e-5) * g2_ref[...]
                    + be2_ref[...], 0.0)
    o_ref[...] = (_dotd(r, wo_ref[...])
                  + bo_ref[...])


def _reg(pooled, wr1, br1, g1, be1, wr2, br2, g2, be2, wo, bo):
    return pl.pallas_call(
        _reg_body,
        out_shape=jax.ShapeDtypeStruct((G, 1), jnp.float32),
    )(pooled, wr1, br1, g1, be1, wr2, br2, g2, be2, wo, bo)


def kernel(x, edge_index, edge_attr, batch, We, be, eps, W1, b1, W2, b2,
           ln_g, ln_b, Wg, bg, Wr1, br1, g1, beta1, Wr2, br2, g2, beta2,
           Wo, bo):
    src = edge_index[0]
    dst = edge_index[1]
    perm = jnp.argsort(dst, stable=True).astype(jnp.int32)
    ssrc2 = src[perm].reshape(E // 128, 128)
    sdst = dst[perm]
    sdst2 = sdst.reshape(E // 128, 128)
    perm2 = perm.reshape(E // 128, 128)
    nws = jnp.array([312 * w for w in range(31)] + [9680, 10000] + [N] * 95,
                    jnp.int32)
    bnd = jnp.searchsorted(sdst, nws).astype(jnp.int32).reshape(1, 128)
    L = We.shape[0]
    h = x
    for l in range(L):
        e = _edge_lin(edge_attr, We[l], be[l].reshape(1, D))
        agg = _gine_sc(h, e, ssrc2, sdst2, perm2, bnd)
        h = _node_mlp(h, agg, (1.0 + eps[l]).reshape(1, 1), W1[l],
                      b1[l].reshape(1, D), W2[l], b2[l].reshape(1, D),
                      ln_g[l].reshape(1, D), ln_b[l].reshape(1, D))
    pooled = _pool(h, batch.reshape(N, 1), Wg, bg.reshape(1, 1))
    return _reg(pooled, Wr1, br1.reshape(1, D), g1.reshape(1, D),
                beta1.reshape(1, D), Wr2, br2.reshape(1, 64),
                g2.reshape(1, 64), beta2.reshape(1, 64), Wo,
                bo.reshape(1, 1))
